# Initial kernel scaffold; baseline (speedup 1.0000x reference)
import jax
import jax.numpy as jnp
from jax.experimental import pallas as pl


def kernel(x, weight, bias):
    raise NotImplementedError("write your pallas kernel here")


# speedup vs baseline: 1.0928x; 1.0928x over previous
import math

import numpy as np
import jax
import jax.numpy as jnp
from jax.experimental import pallas as pl
from jax.experimental.pallas import tpu as pltpu

_NR = 3
_CUTOFF = 0.01
_DOM = 2.0
_EPS = 1e-9
_H = 512
_W = 512


def _psi_q_table():
    dh = _DOM / _H
    dw = _DOM / _W
    off = math.floor(_CUTOFF / dh)
    p = 2 * off + 1
    ys = (np.arange(p) - off) * dh
    xs = (np.arange(p) - off) * dw
    yy, xx = np.meshgrid(ys, xs, indexing='ij')
    r = np.sqrt(yy ** 2 + xx ** 2).reshape(-1)
    dr = _CUTOFF / _NR
    k = np.arange(_NR).reshape(-1, 1)
    vals = np.maximum(0.0, 1.0 - np.abs(r[None, :] - k * dr) / dr)
    vals = np.where(r[None, :] <= _CUTOFF, vals, 0.0)
    q = dh * dw
    for ik in range(math.ceil(_NR / 2)):
        vals[ik] = vals[ik] / (np.sum(vals[ik] * q) + _EPS)
    return (vals * q).astype(np.float32).reshape(_NR, p, p), p


_PSIQ, _P = _psi_q_table()
_PAD = (_P - 1) // 2
_TAPS = []
for _u in range(_P):
    for _v in range(_P):
        nz = [(k, float(_PSIQ[k, _u, _v])) for k in range(_NR)
              if _PSIQ[k, _u, _v] != 0.0]
        if nz:
            _TAPS.append((_u, _v, nz))


def _body(w_ref, b_ref, x_ref, o_ref):
    c = pl.program_id(1)
    xb = x_ref[0, 0]
    zc = jnp.zeros((_H, _PAD), jnp.float32)
    pc = jnp.concatenate([zc, xb, zc], axis=1)
    zr = jnp.zeros((_PAD, _W + 2 * _PAD), jnp.float32)
    p = jnp.concatenate([zr, pc, zr], axis=0)

    w = [w_ref[c, k] for k in range(_NR)]
    acc = None
    for u, v, nz in _TAPS:
        kv = None
        for k, val in nz:
            term = w[k] * val
            kv = term if kv is None else kv + term
        sl = jax.lax.slice(p, (u, v), (u + _H, v + _W))
        t = kv * sl
        acc = t if acc is None else acc + t
    o_ref[0, 0] = acc + b_ref[c, 0]


def kernel(x, weight, bias):
    B, C, H, W = x.shape
    w2d = weight.reshape(C, _NR)
    b2d = bias.reshape(C, 1)
    return pl.pallas_call(
        _body,
        grid=(B, C),
        in_specs=[
            pl.BlockSpec(memory_space=pltpu.SMEM),
            pl.BlockSpec(memory_space=pltpu.SMEM),
            pl.BlockSpec((1, 1, H, W), lambda b, c: (b, c, 0, 0)),
        ],
        out_specs=pl.BlockSpec((1, 1, H, W), lambda b, c: (b, c, 0, 0)),
        out_shape=jax.ShapeDtypeStruct((B, C, H, W), jnp.float32),
    )(w2d, b2d, x)


# speedup vs baseline: 1.1510x; 1.0533x over previous
import math

import numpy as np
import jax
import jax.numpy as jnp
from jax.experimental import pallas as pl
from jax.experimental.pallas import tpu as pltpu

_NR = 3
_CUTOFF = 0.01
_DOM = 2.0
_EPS = 1e-9
_H = 512
_W = 512


def _psi_q_table():
    dh = _DOM / _H
    dw = _DOM / _W
    off = math.floor(_CUTOFF / dh)
    p = 2 * off + 1
    ys = (np.arange(p) - off) * dh
    xs = (np.arange(p) - off) * dw
    yy, xx = np.meshgrid(ys, xs, indexing='ij')
    r = np.sqrt(yy ** 2 + xx ** 2).reshape(-1)
    dr = _CUTOFF / _NR
    k = np.arange(_NR).reshape(-1, 1)
    vals = np.maximum(0.0, 1.0 - np.abs(r[None, :] - k * dr) / dr)
    vals = np.where(r[None, :] <= _CUTOFF, vals, 0.0)
    q = dh * dw
    for ik in range(math.ceil(_NR / 2)):
        vals[ik] = vals[ik] / (np.sum(vals[ik] * q) + _EPS)
    return (vals * q).astype(np.float32).reshape(_NR, p, p), p


_PSIQ, _P = _psi_q_table()
_PAD = (_P - 1) // 2
_TAPS = []
for _u in range(_P):
    for _v in range(_P):
        nz = [(k, float(_PSIQ[k, _u, _v])) for k in range(_NR)
              if _PSIQ[k, _u, _v] != 0.0]
        if nz:
            _TAPS.append((_u, _v, nz))


_STRIP = 32
_PW = _W + 2 * _PAD


def _body(w_ref, b_ref, x_ref, o_ref, p_ref):
    c = pl.program_id(1)
    xb = x_ref[0, 0]
    zc = jnp.zeros((_H, _PAD), jnp.float32)
    pc = jnp.concatenate([zc, xb, zc], axis=1)
    zr = jnp.zeros((_PAD, _PW), jnp.float32)
    p_ref[:, :] = jnp.concatenate([zr, pc, zr], axis=0)

    w = [w_ref[c, k] for k in range(_NR)]
    kvs = []
    for u, v, nz in _TAPS:
        kv = None
        for k, val in nz:
            term = w[k] * val
            kv = term if kv is None else kv + term
        kvs.append((u, v, kv))

    bc = b_ref[c, 0]
    for s in range(0, _H, _STRIP):
        acc = None
        for u, v, kv in kvs:
            sl = p_ref[pl.ds(s + u, _STRIP), pl.ds(v, _W)]
            t = kv * sl
            acc = t if acc is None else acc + t
        o_ref[0, 0, pl.ds(s, _STRIP), :] = acc + bc


def kernel(x, weight, bias):
    B, C, H, W = x.shape
    w2d = weight.reshape(C, _NR)
    b2d = bias.reshape(C, 1)
    return pl.pallas_call(
        _body,
        grid=(B, C),
        in_specs=[
            pl.BlockSpec(memory_space=pltpu.SMEM),
            pl.BlockSpec(memory_space=pltpu.SMEM),
            pl.BlockSpec((1, 1, H, W), lambda b, c: (b, c, 0, 0)),
        ],
        out_specs=pl.BlockSpec((1, 1, H, W), lambda b, c: (b, c, 0, 0)),
        out_shape=jax.ShapeDtypeStruct((B, C, H, W), jnp.float32),
        scratch_shapes=[pltpu.VMEM((_H + 2 * _PAD, _PW), jnp.float32)],
    )(w2d, b2d, x)


# speedup vs baseline: 3.3203x; 2.8846x over previous
import math

import numpy as np
import jax
import jax.numpy as jnp
from jax.experimental import pallas as pl
from jax.experimental.pallas import tpu as pltpu

_NR = 3
_CUTOFF = 0.01
_DOM = 2.0
_EPS = 1e-9
_H = 512
_W = 512


def _psi_q_table():
    dh = _DOM / _H
    dw = _DOM / _W
    off = math.floor(_CUTOFF / dh)
    p = 2 * off + 1
    ys = (np.arange(p) - off) * dh
    xs = (np.arange(p) - off) * dw
    yy, xx = np.meshgrid(ys, xs, indexing='ij')
    r = np.sqrt(yy ** 2 + xx ** 2).reshape(-1)
    dr = _CUTOFF / _NR
    k = np.arange(_NR).reshape(-1, 1)
    vals = np.maximum(0.0, 1.0 - np.abs(r[None, :] - k * dr) / dr)
    vals = np.where(r[None, :] <= _CUTOFF, vals, 0.0)
    q = dh * dw
    for ik in range(math.ceil(_NR / 2)):
        vals[ik] = vals[ik] / (np.sum(vals[ik] * q) + _EPS)
    return (vals * q).astype(np.float32).reshape(_NR, p, p), p


_PSIQ, _P = _psi_q_table()
_PAD = (_P - 1) // 2
assert np.allclose(_PSIQ, _PSIQ[:, ::-1, :]) and np.allclose(_PSIQ, _PSIQ[:, :, ::-1])

_STRIP = 32
_ROWS = _H + 2 * _PAD + 4


def _body(w_ref, b_ref, x_ref, o_ref, e_ref):
    c = pl.program_id(1)
    xb = x_ref[0, 0]
    zc = jnp.zeros((_H, _PAD), jnp.float32)
    pc = jnp.concatenate([zc, xb, zc], axis=1)
    zr = jnp.zeros((_PAD, _W), jnp.float32)
    zr2 = jnp.zeros((_ROWS - _H - _PAD, _W), jnp.float32)

    e2 = (jax.lax.slice(pc, (0, 0), (_H, _W))
          + jax.lax.slice(pc, (0, 4), (_H, 4 + _W)))
    e1 = (jax.lax.slice(pc, (0, 1), (_H, 1 + _W))
          + jax.lax.slice(pc, (0, 3), (_H, 3 + _W)))
    l2 = jax.lax.slice(pc, (0, 2), (_H, 2 + _W))
    e_ref[0] = jnp.concatenate([zr, e2, zr2], axis=0)
    e_ref[1] = jnp.concatenate([zr, e1, zr2], axis=0)
    e_ref[2] = jnp.concatenate([zr, l2, zr2], axis=0)

    w = [w_ref[c, k] for k in range(_NR)]
    kc = [[None] * 3 for _ in range(3)]
    for u in range(3):
        for col in range(3):
            kv = None
            for k in range(_NR):
                val = float(_PSIQ[k, u, col])
                if val != 0.0:
                    t = w[k] * val
                    kv = t if kv is None else kv + t
            kc[u][col] = kv

    bc = b_ref[c, 0]
    for s in range(0, _H, _STRIP):
        blk = [e_ref[i, pl.ds(s, _STRIP + 8), :] for i in range(3)]
        pu = []
        for u in range(3):
            acc = None
            for col in range(3):
                if kc[u][col] is None:
                    continue
                t = kc[u][col] * blk[col]
                acc = t if acc is None else acc + t
            pu.append(acc)
        out = None
        for u in range(5):
            src = pu[u if u <= 2 else 4 - u]
            sl = jax.lax.slice(src, (u, 0), (u + _STRIP, _W))
            out = sl if out is None else out + sl
        o_ref[0, 0, pl.ds(s, _STRIP), :] = out + bc


def kernel(x, weight, bias):
    B, C, H, W = x.shape
    w2d = weight.reshape(C, _NR)
    b2d = bias.reshape(C, 1)
    return pl.pallas_call(
        _body,
        grid=(B, C),
        in_specs=[
            pl.BlockSpec(memory_space=pltpu.SMEM),
            pl.BlockSpec(memory_space=pltpu.SMEM),
            pl.BlockSpec((1, 1, H, W), lambda b, c: (b, c, 0, 0)),
        ],
        out_specs=pl.BlockSpec((1, 1, H, W), lambda b, c: (b, c, 0, 0)),
        out_shape=jax.ShapeDtypeStruct((B, C, H, W), jnp.float32),
        scratch_shapes=[pltpu.VMEM((3, _ROWS, _W), jnp.float32)],
    )(w2d, b2d, x)


# speedup vs baseline: 7.1068x; 2.1404x over previous
import math

import numpy as np
import jax
import jax.numpy as jnp
from jax.experimental import pallas as pl
from jax.experimental.pallas import tpu as pltpu

_NR = 3
_CUTOFF = 0.01
_DOM = 2.0
_EPS = 1e-9
_H = 512
_W = 512


def _psi_q_table():
    dh = _DOM / _H
    dw = _DOM / _W
    off = math.floor(_CUTOFF / dh)
    p = 2 * off + 1
    ys = (np.arange(p) - off) * dh
    xs = (np.arange(p) - off) * dw
    yy, xx = np.meshgrid(ys, xs, indexing='ij')
    r = np.sqrt(yy ** 2 + xx ** 2).reshape(-1)
    dr = _CUTOFF / _NR
    k = np.arange(_NR).reshape(-1, 1)
    vals = np.maximum(0.0, 1.0 - np.abs(r[None, :] - k * dr) / dr)
    vals = np.where(r[None, :] <= _CUTOFF, vals, 0.0)
    q = dh * dw
    for ik in range(math.ceil(_NR / 2)):
        vals[ik] = vals[ik] / (np.sum(vals[ik] * q) + _EPS)
    return (vals * q).astype(np.float32).reshape(_NR, p, p), p


_PSIQ, _P = _psi_q_table()
_PAD = (_P - 1) // 2
assert np.allclose(_PSIQ, _PSIQ[:, ::-1, :]) and np.allclose(_PSIQ, _PSIQ[:, :, ::-1])

_STRIP = 64


_CT = jnp.bfloat16


def _xwin(xb, a, n):
    lo = a - _PAD
    hi = lo + n
    top = max(0, -lo)
    bot = max(0, hi - _H)
    core = jax.lax.slice(xb, (max(0, lo), 0), (max(0, lo) + n - top - bot, _W))
    parts = []
    if top:
        parts.append(jnp.zeros((top, _W), _CT))
    parts.append(core)
    if bot:
        parts.append(jnp.zeros((bot, _W), _CT))
    return jnp.concatenate(parts, axis=0) if len(parts) > 1 else parts[0]


def _colpair(w, d, rows):
    zero = jnp.array(0, _CT)
    left = jax.lax.pad(jax.lax.slice(w, (0, d), (rows, _W)),
                       zero, [(0, 0, 0), (0, d, 0)])
    right = jax.lax.pad(jax.lax.slice(w, (0, 0), (rows, _W - d)),
                        zero, [(0, 0, 0), (d, 0, 0)])
    return left + right


_CPB = 8


def _body(w_ref, b_ref, x_ref, o_ref):
    c0 = pl.program_id(1) * _CPB
    nrows = _STRIP + 2 * _PAD
    for ch in range(_CPB):
        c = c0 + ch
        w = [w_ref[c, k] for k in range(_NR)]
        kc = [[None] * 3 for _ in range(3)]
        for u in range(3):
            for col in range(3):
                kv = None
                for k in range(_NR):
                    val = float(_PSIQ[k, u, col])
                    if val != 0.0:
                        t = w[k] * val
                        kv = t if kv is None else kv + t
                kc[u][col] = None if kv is None else kv.astype(_CT)

        bc = b_ref[c, 0].astype(_CT)
        xb16 = x_ref[0, ch].astype(_CT)
        for s in range(0, _H, _STRIP):
            xw = _xwin(xb16, s, nrows)
            cols = [_colpair(xw, 2, nrows), _colpair(xw, 1, nrows), xw]
            pu = []
            for u in range(3):
                acc = None
                for col in range(3):
                    if kc[u][col] is None:
                        continue
                    t = kc[u][col] * cols[col]
                    acc = t if acc is None else acc + t
                pu.append(acc)
            p0, p1, p2 = pu
            t_odd = (jax.lax.slice(p1, (0, 0), (nrows - 2, _W))
                     + jax.lax.slice(p1, (2, 0), (nrows, _W)))
            inner = (jax.lax.slice(p2, (0, 0), (nrows - 2, _W))
                     + jax.lax.slice(p0, (2, 0), (nrows, _W)))
            s_even = (jax.lax.slice(p0, (0, 0), (_STRIP, _W))
                      + jax.lax.slice(inner, (2, 0), (2 + _STRIP, _W)))
            out = s_even + jax.lax.slice(t_odd, (1, 0), (1 + _STRIP, _W)) + bc
            o_ref[0, ch, pl.ds(s, _STRIP), :] = out.astype(jnp.float32)


def kernel(x, weight, bias):
    B, C, H, W = x.shape
    w2d = weight.reshape(C, _NR)
    b2d = bias.reshape(C, 1)
    return pl.pallas_call(
        _body,
        grid=(B, C // _CPB),
        in_specs=[
            pl.BlockSpec(memory_space=pltpu.SMEM),
            pl.BlockSpec(memory_space=pltpu.SMEM),
            pl.BlockSpec((1, _CPB, H, W), lambda b, c: (b, c, 0, 0)),
        ],
        out_specs=pl.BlockSpec((1, _CPB, H, W), lambda b, c: (b, c, 0, 0)),
        out_shape=jax.ShapeDtypeStruct((B, C, H, W), jnp.float32),
    )(w2d, b2d, x)
